# Initial kernel scaffold; baseline (speedup 1.0000x reference)
import functools

import jax
import jax.numpy as jnp
from jax import lax
from jax.experimental import pallas as pl
from jax.experimental.pallas import tpu as pltpu
from jax.experimental.pallas import tpu_sc as plsc

N = 10000
E = 320000
DF = 128
D = 32
OUT = 300
G = 256
BN_EPS = 1e-5

NC, NS = 2, 16
NW = NC * NS
P = 10240
ZROW = N
TRASH = N + 8
CHUNK = 128
NCHUNK = 80
EPW = NCHUNK * CHUNK
EPAD = NW * EPW
RPS = P // NS


def _sc_segment_sum(p_tab, edges):
    mesh = plsc.VectorSubcoreMesh(core_axis_name="c", subcore_axis_name="s")

    @functools.partial(
        pl.kernel,
        mesh=mesh,
        out_type=jax.ShapeDtypeStruct((NC, P, D), jnp.float32),
        scratch_types=[
            pltpu.VMEM((NCHUNK, CHUNK), jnp.int32),
            pltpu.VMEM((NCHUNK, CHUNK), jnp.int32),
            pltpu.VMEM((CHUNK, D), jnp.float32),
            pltpu.VMEM((CHUNK, D), jnp.float32),
            pltpu.VMEM((RPS, D), jnp.float32),
            pltpu.VMEM_SHARED((P, D), jnp.float32),
            pltpu.VMEM_SHARED((P, D), jnp.float32),
            pltpu.SemaphoreType.DMA,
            pltpu.SemaphoreType.DMA,
        ],
    )
    def seg_kernel(p_hbm, e_hbm, out_hbm, src_v, dst_v, rows_a, rows_b,
                   zbuf, p_sh, agg_sh, sem_a, sem_b):
        c = lax.axis_index("c")
        s = lax.axis_index("s")
        wid = c * NS + s
        base = s * RPS

        def _zero_row(i, carry):
            zbuf[i, pl.ds(0, 16)] = jnp.zeros((16,), jnp.float32)
            zbuf[i, pl.ds(16, 16)] = jnp.zeros((16,), jnp.float32)
            return carry
        lax.fori_loop(0, RPS, _zero_row, 0)

        pltpu.sync_copy(p_hbm.at[pl.ds(base, RPS)], p_sh.at[pl.ds(base, RPS)])
        pltpu.sync_copy(zbuf, agg_sh.at[pl.ds(base, RPS)])

        pltpu.sync_copy(e_hbm.at[0, wid], src_v)
        pltpu.sync_copy(e_hbm.at[1, wid], dst_v)
        plsc.subcore_barrier()

        def gather(j, buf, sem):
            return pltpu.make_async_copy(p_sh.at[src_v.at[j]], buf, sem)

        gather(0, rows_a, sem_a).start()

        def chunk_pair(j2, carry):
            j = j2 * 2
            gather(j + 1, rows_b, sem_b).start()
            gather(j, rows_a, sem_a).wait()
            pltpu.sync_copy(rows_a, agg_sh.at[dst_v.at[j]], add=True)

            @pl.when(j2 < NCHUNK // 2 - 1)
            def _():
                gather(j + 2, rows_a, sem_a).start()

            gather(j + 1, rows_b, sem_b).wait()
            pltpu.sync_copy(rows_b, agg_sh.at[dst_v.at[j + 1]], add=True)
            return carry

        lax.fori_loop(0, NCHUNK // 2, chunk_pair, 0)

        plsc.subcore_barrier()
        pltpu.sync_copy(agg_sh.at[pl.ds(base, RPS)],
                        out_hbm.at[c, pl.ds(base, RPS)])

    return seg_kernel(p_tab, edges)


def _tc_project(x_pad, w1):
    def body(x_ref, w_ref, o_ref):
        o_ref[...] = jnp.dot(x_ref[...], w_ref[...],
                             preferred_element_type=jnp.float32)
    return pl.pallas_call(
        body,
        out_shape=jax.ShapeDtypeStruct((P, D), jnp.float32),
    )(x_pad, w1)


def _layer_body(p_ref, parts_ref, b1_ref, w2_ref, b2_ref, g_ref, be_ref):
    u = p_ref[...] + parts_ref[0] + parts_ref[1] + b1_ref[...]
    a = jnp.maximum(u, 0.0)
    z = jnp.dot(a, w2_ref[...], preferred_element_type=jnp.float32) + b2_ref[...]
    z = jnp.maximum(z, 0.0)
    valid = lax.broadcasted_iota(jnp.int32, (P, D), 0) < N
    z = jnp.where(valid, z, 0.0)
    mean = jnp.sum(z, axis=0, keepdims=True) * (1.0 / N)
    zc = jnp.where(valid, z - mean, 0.0)
    var = jnp.sum(zc * zc, axis=0, keepdims=True) * (1.0 / N)
    h = g_ref[...] * zc * lax.rsqrt(var + BN_EPS) + be_ref[...]
    return jnp.where(valid, h, 0.0)


def _tc_layer(p, parts, b1, w2, b2, gamma, beta, w1_next):
    def body(p_ref, parts_ref, b1_ref, w2_ref, b2_ref, g_ref, be_ref,
             w1n_ref, o_ref):
        h = _layer_body(p_ref, parts_ref, b1_ref, w2_ref, b2_ref, g_ref, be_ref)
        o_ref[...] = jnp.dot(h, w1n_ref[...], preferred_element_type=jnp.float32)
    return pl.pallas_call(
        body,
        out_shape=jax.ShapeDtypeStruct((P, D), jnp.float32),
    )(p, parts, b1.reshape(1, D), w2, b2.reshape(1, D),
      gamma.reshape(1, D), beta.reshape(1, D), w1_next)


def _tc_final(p, parts, b1, w2, b2, gamma, beta, batch_pad, fc_w, fc_b):
    def body(p_ref, parts_ref, b1_ref, w2_ref, b2_ref, g_ref, be_ref,
             bt_ref, fcw_ref, fcb_ref, o_ref):
        h = _layer_body(p_ref, parts_ref, b1_ref, w2_ref, b2_ref, g_ref, be_ref)
        gid = lax.broadcasted_iota(jnp.int32, (G, P), 0)
        onehot = jnp.where(gid == bt_ref[...], 1.0, 0.0)
        pooled = jnp.dot(onehot, h, preferred_element_type=jnp.float32)
        o_ref[...] = jnp.maximum(
            jnp.dot(pooled, fcw_ref[...], preferred_element_type=jnp.float32)
            + fcb_ref[...], 0.0)
    return pl.pallas_call(
        body,
        out_shape=jax.ShapeDtypeStruct((G, OUT), jnp.float32),
    )(p, parts, b1.reshape(1, D), w2, b2.reshape(1, D),
      gamma.reshape(1, D), beta.reshape(1, D), batch_pad, fc_w,
      fc_b.reshape(1, OUT))


def kernel(x, edge_index, batch, params):
    layers = params["layers"]

    x_pad = jnp.pad(x, ((0, P - N), (0, 0)))
    pad_e = EPAD - E
    src_pad = jnp.concatenate(
        [edge_index[0], jnp.full((pad_e,), ZROW, jnp.int32)])
    dst_pad = jnp.concatenate(
        [edge_index[1], jnp.full((pad_e,), TRASH, jnp.int32)])
    edges = jnp.stack([src_pad, dst_pad]).reshape(2, NW, NCHUNK, CHUNK)
    batch_pad = jnp.concatenate(
        [batch, jnp.full((P - N,), G, jnp.int32)]).reshape(1, P)

    p = _tc_project(x_pad, layers[0]["mlp"][0])
    for l in range(5):
        parts = _sc_segment_sum(p, edges)
        w1, b1, w2, b2 = layers[l]["mlp"]
        if l < 4:
            p = _tc_layer(p, parts, b1, w2, b2,
                          layers[l]["gamma"], layers[l]["beta"],
                          layers[l + 1]["mlp"][0])
        else:
            out = _tc_final(p, parts, b1, w2, b2,
                            layers[l]["gamma"], layers[l]["beta"],
                            batch_pad, params["fc_w"], params["fc_b"])
    return out


# speedup vs baseline: 12.0782x; 12.0782x over previous
import functools

import jax
import jax.numpy as jnp
from jax import lax
from jax.experimental import pallas as pl
from jax.experimental.pallas import tpu as pltpu
from jax.experimental.pallas import tpu_sc as plsc

N = 10000
E = 320000
DF = 128
D = 32
OUT = 300
G = 256
BN_EPS = 1e-5

NC, NS = 2, 16
NW = NC * NS
P = 10240
ZROW = N
TRASH = N + 8
CHUNK = 128
NCHUNK = 80
EPAD = NW * NCHUNK * CHUNK

PG = 512
GTRASH = G + 8
CHUNK_P = 64
NCH_P = P // (NW * CHUNK_P)


def _make_sc_segsum(width, src_rows, dst_rows, nchunk):
    mesh = plsc.VectorSubcoreMesh(core_axis_name="c", subcore_axis_name="s")
    spr = src_rows // NS
    dpr = dst_rows // NS

    @functools.partial(
        pl.kernel,
        mesh=mesh,
        out_type=jax.ShapeDtypeStruct((NC, dst_rows, width), jnp.float32),
        scratch_types=[
            pltpu.VMEM((nchunk, CHUNK), jnp.int32),
            pltpu.VMEM((nchunk, CHUNK), jnp.int32),
            pltpu.VMEM((CHUNK, width), jnp.float32),
            pltpu.VMEM((CHUNK, width), jnp.float32),
            pltpu.VMEM((dpr, width), jnp.float32),
            pltpu.VMEM_SHARED((src_rows, width), jnp.float32),
            pltpu.VMEM_SHARED((dst_rows, width), jnp.float32),
            pltpu.SemaphoreType.DMA,
            pltpu.SemaphoreType.DMA,
        ],
        compiler_params=pltpu.CompilerParams(use_tc_tiling_on_sc=False),
    )
    def seg_kernel(tab_hbm, e_hbm, out_hbm, src_v, dst_v, rows_a, rows_b,
                   zbuf, tab_sh, agg_sh, sem_a, sem_b):
        c = lax.axis_index("c")
        s = lax.axis_index("s")
        wid = c * NS + s

        def _zero_row(i, carry):
            for w16 in range(width // 16):
                zbuf[i, pl.ds(w16 * 16, 16)] = jnp.zeros((16,), jnp.float32)
            return carry
        lax.fori_loop(0, dpr, _zero_row, 0)

        pltpu.sync_copy(tab_hbm.at[pl.ds(s * spr, spr)],
                        tab_sh.at[pl.ds(s * spr, spr)])
        pltpu.sync_copy(zbuf, agg_sh.at[pl.ds(s * dpr, dpr)])

        pltpu.sync_copy(e_hbm.at[0, wid], src_v)
        pltpu.sync_copy(e_hbm.at[1, wid], dst_v)
        plsc.subcore_barrier()

        def gather(j, buf, sem):
            return pltpu.make_async_copy(tab_sh.at[src_v.at[j]], buf, sem)

        gather(0, rows_a, sem_a).start()

        def chunk_pair(j2, carry):
            j = j2 * 2
            gather(j + 1, rows_b, sem_b).start()
            gather(j, rows_a, sem_a).wait()
            pltpu.sync_copy(rows_a, agg_sh.at[dst_v.at[j]], add=True)

            @pl.when(j2 < nchunk // 2 - 1)
            def _():
                gather(j + 2, rows_a, sem_a).start()

            gather(j + 1, rows_b, sem_b).wait()
            pltpu.sync_copy(rows_b, agg_sh.at[dst_v.at[j + 1]], add=True)
            return carry

        lax.fori_loop(0, nchunk // 2, chunk_pair, 0)

        plsc.subcore_barrier()
        pltpu.sync_copy(agg_sh.at[pl.ds(s * dpr, dpr)],
                        out_hbm.at[c, pl.ds(s * dpr, dpr)])

    return seg_kernel


_sc_edges_d = _make_sc_segsum(D, P, P, NCHUNK)


def _sc_pool(h_tab, pool_dst):
    mesh = plsc.VectorSubcoreMesh(core_axis_name="c", subcore_axis_name="s")
    dpr = PG // NS

    @functools.partial(
        pl.kernel,
        mesh=mesh,
        out_type=jax.ShapeDtypeStruct((NC, PG, D), jnp.float32),
        scratch_types=[
            pltpu.VMEM((NCH_P, CHUNK_P), jnp.int32),
            pltpu.VMEM((CHUNK_P, D), jnp.float32),
            pltpu.VMEM((dpr, D), jnp.float32),
            pltpu.VMEM_SHARED((PG, D), jnp.float32),
        ],
        compiler_params=pltpu.CompilerParams(use_tc_tiling_on_sc=False),
    )
    def pool_kernel(h_hbm, pd_hbm, out_hbm, dst_v, rows_v, zbuf, agg_sh):
        c = lax.axis_index("c")
        s = lax.axis_index("s")
        wid = c * NS + s

        def _zero_row(i, carry):
            zbuf[i, pl.ds(0, 16)] = jnp.zeros((16,), jnp.float32)
            zbuf[i, pl.ds(16, 16)] = jnp.zeros((16,), jnp.float32)
            return carry
        lax.fori_loop(0, dpr, _zero_row, 0)

        pltpu.sync_copy(zbuf, agg_sh.at[pl.ds(s * dpr, dpr)])
        pltpu.sync_copy(pd_hbm.at[wid], dst_v)
        plsc.subcore_barrier()

        base = wid * (NCH_P * CHUNK_P)
        for j in range(NCH_P):
            pltpu.sync_copy(h_hbm.at[pl.ds(base + j * CHUNK_P, CHUNK_P)],
                            rows_v)
            pltpu.sync_copy(rows_v, agg_sh.at[dst_v.at[j]], add=True)

        plsc.subcore_barrier()
        pltpu.sync_copy(agg_sh.at[pl.ds(s * dpr, dpr)],
                        out_hbm.at[c, pl.ds(s * dpr, dpr)])

    return pool_kernel(h_tab, pool_dst)


def _bn_mlp(u, w1_ref, b1_ref, w2_ref, b2_ref, g_ref, be_ref):
    z = jnp.dot(jnp.maximum(jnp.dot(u, w1_ref[...],
                                    preferred_element_type=jnp.float32)
                            + b1_ref[...], 0.0),
                w2_ref[...], preferred_element_type=jnp.float32) + b2_ref[...]
    z = jnp.maximum(z, 0.0)
    valid = lax.broadcasted_iota(jnp.int32, (P, D), 0) < N
    z = jnp.where(valid, z, 0.0)
    mean = jnp.sum(z, axis=0, keepdims=True) * (1.0 / N)
    zc = jnp.where(valid, z - mean, 0.0)
    var = jnp.sum(zc * zc, axis=0, keepdims=True) * (1.0 / N)
    h = g_ref[...] * zc / jnp.sqrt(var + BN_EPS) + be_ref[...]
    return jnp.where(valid, h, 0.0)


def _tc_layer1(x_pad, parts4, w1, b1, w2, b2, gamma, beta):
    parts = jnp.concatenate(parts4, axis=2)
    def body(x_ref, parts_ref, w1_ref, b1_ref,
             w2_ref, b2_ref, g_ref, be_ref, o_ref):
        u = x_ref[...] + parts_ref[0] + parts_ref[1]
        o_ref[...] = _bn_mlp(u, w1_ref, b1_ref, w2_ref, b2_ref, g_ref, be_ref)
    return pl.pallas_call(
        body,
        out_shape=jax.ShapeDtypeStruct((P, D), jnp.float32),
    )(x_pad, parts, w1, b1.reshape(1, D), w2, b2.reshape(1, D),
      gamma.reshape(1, D), beta.reshape(1, D))


def _tc_layer(h, parts, w1, b1, w2, b2, gamma, beta):
    def body(h_ref, parts_ref, w1_ref, b1_ref, w2_ref, b2_ref,
             g_ref, be_ref, o_ref):
        u = h_ref[...] + parts_ref[0] + parts_ref[1]
        o_ref[...] = _bn_mlp(u, w1_ref, b1_ref, w2_ref, b2_ref, g_ref, be_ref)
    return pl.pallas_call(
        body,
        out_shape=jax.ShapeDtypeStruct((P, D), jnp.float32),
    )(h, parts, w1, b1.reshape(1, D), w2, b2.reshape(1, D),
      gamma.reshape(1, D), beta.reshape(1, D))


def _tc_fc(pool_parts, fc_w, fc_b):
    def body(pp_ref, w_ref, b_ref, o_ref):
        pooled = pp_ref[0, :G, :] + pp_ref[1, :G, :]
        o_ref[...] = jnp.maximum(
            jnp.dot(pooled, w_ref[...], preferred_element_type=jnp.float32)
            + b_ref[...], 0.0)
    return pl.pallas_call(
        body,
        out_shape=jax.ShapeDtypeStruct((G, OUT), jnp.float32),
    )(pool_parts, fc_w, fc_b.reshape(1, OUT))


def kernel(x, edge_index, batch, params):
    layers = params["layers"]

    x_pad = jnp.pad(x, ((0, P - N), (0, 0)))
    pad_e = EPAD - E
    src_pad = jnp.concatenate(
        [edge_index[0], jnp.full((pad_e,), ZROW, jnp.int32)])
    dst_pad = jnp.concatenate(
        [edge_index[1], jnp.full((pad_e,), TRASH, jnp.int32)])
    edges = jnp.stack([src_pad, dst_pad]).reshape(2, NW, NCHUNK, CHUNK)

    pool_dst = jnp.concatenate(
        [batch, jnp.full((P - N,), GTRASH, jnp.int32)]).reshape(
        NW, NCH_P, CHUNK_P)

    h = x_pad
    for l in range(5):
        w1, b1, w2, b2 = layers[l]["mlp"]
        gamma, beta = layers[l]["gamma"], layers[l]["beta"]
        if l == 0:
            parts4 = [_sc_edges_d(h[:, k * D:(k + 1) * D], edges)
                      for k in range(DF // D)]
            h = _tc_layer1(h, parts4, w1, b1, w2, b2, gamma, beta)
        else:
            parts = _sc_edges_d(h, edges)
            h = _tc_layer(h, parts, w1, b1, w2, b2, gamma, beta)

    pool_parts = _sc_pool(h, pool_dst)
    return _tc_fc(pool_parts, params["fc_w"], params["fc_b"])
